# Initial kernel scaffold; baseline (speedup 1.0000x reference)
import jax
import jax.numpy as jnp
from jax.experimental import pallas as pl


def kernel(x, edge_index, params):
    raise NotImplementedError("write your pallas kernel here")


# speedup vs baseline: 20.9026x; 20.9026x over previous
import functools

import jax
import jax.numpy as jnp
from jax import lax
from jax.experimental import pallas as pl
from jax.experimental.pallas import tpu as pltpu
from jax.experimental.pallas import tpu_sc as plsc

T, N, F, E = 8, 10000, 128, 320000
H, L, LH = 64, 32, 64

NC, NS = 2, 16
NW = NC * NS
EW = E // NW
CH = 80
NCHUNK = EW // CH
NP = 10240
ROWS_PER_TILE = NP // NS
CPR = 128
NCP = ROWS_PER_TILE // CPR
ZPR = 32
NZP = ROWS_PER_TILE // ZPR
SEG = 2000
NSEG = EW // SEG
CPS = SEG // CH
WP = 128
NV_SCALE = 5

_PREC = lax.Precision.HIGHEST


@functools.lru_cache(maxsize=None)
def _make_gat_edges(TB: int):
    mesh = plsc.VectorSubcoreMesh(core_axis_name="c", subcore_axis_name="s")

    @functools.partial(
        pl.kernel,
        out_type=jax.ShapeDtypeStruct((TB, NC, NP, WP), jnp.float32),
        mesh=mesh,
        compiler_params=pltpu.CompilerParams(needs_layout_passes=False),
        scratch_types=[
            pltpu.VMEM_SHARED((NP, WP), jnp.float32),
            pltpu.VMEM((N,), jnp.float32),
            pltpu.VMEM((N,), jnp.float32),
            pltpu.VMEM((SEG,), jnp.int32),
            pltpu.VMEM((SEG,), jnp.int32),
            pltpu.VMEM((CH,), jnp.int32),
            pltpu.VMEM((CH,), jnp.int32),
            pltpu.VMEM((CH, WP), jnp.float32),
            pltpu.VMEM((ZPR, WP), jnp.float32),
            pltpu.SemaphoreType.DMA,
        ],
    )
    def gat_edges(hpad_hbm, sd_hbm, esrc_hbm, edst_hbm, acc_out,
                  acc_sh, s_v, d_v, srcs_v, dsts_v, srci_v, dsti_v,
                  rows_v, zrows_v, sem):
        c = lax.axis_index("c")
        s = lax.axis_index("s")
        wid = c * NS + s
        row0 = s * ROWS_PER_TILE

        def _zr(r, _):
            for v in range(WP // 16):
                zrows_v[r, pl.ds(v * 16, 16)] = jnp.zeros((16,), jnp.float32)
            return 0
        lax.fori_loop(0, ZPR, _zr, 0)

        def per_tb(tb, _):
            for z in range(NZP):
                pltpu.sync_copy(
                    zrows_v, acc_sh.at[pl.ds(row0 + z * ZPR, ZPR)])
            pltpu.sync_copy(sd_hbm.at[tb, 0], s_v)
            pltpu.sync_copy(sd_hbm.at[tb, 1], d_v)
            plsc.subcore_barrier()

            base = tb * N

            def strip(g, _):
                e0 = wid * EW + g * SEG
                pltpu.sync_copy(esrc_hbm.at[pl.ds(e0, SEG)], srcs_v)
                pltpu.sync_copy(edst_hbm.at[pl.ds(e0, SEG)], dsts_v)

                def p2(ci, _):
                    exs = []
                    for w in range(CH // 16):
                        esl = pl.ds(ci * CH + w * 16, 16)
                        srcv = srcs_v[esl]
                        dstv = dsts_v[esl]
                        sv = plsc.load_gather(s_v, [srcv])
                        dv = plsc.load_gather(d_v, [dstv])
                        logit = sv + dv
                        e = jnp.maximum(logit, 0.2 * logit)
                        exs.append(jnp.exp(e))
                        srci_v[pl.ds(w * 16, 16)] = srcv + base
                        dsti_v[pl.ds(w * 16, 16)] = dstv
                    pltpu.async_copy(
                        hpad_hbm.at[srci_v], rows_v, sem).wait()
                    for w in range(CH // 16):
                        exv = exs[w]
                        for k2 in range(16):
                            exb = jnp.full((16,), exv[k2], jnp.float32)
                            row = w * 16 + k2
                            for v in range(NV_SCALE):
                                rows_v[row, pl.ds(v * 16, 16)] = (
                                    rows_v[row, pl.ds(v * 16, 16)] * exb)
                    pltpu.sync_copy(rows_v, acc_sh.at[dsti_v], add=True)
                    return 0
                lax.fori_loop(0, CPS, p2, 0)
                return 0
            lax.fori_loop(0, NSEG, strip, 0)
            plsc.subcore_barrier()

            for z in range(NCP):
                rsl = pl.ds(row0 + z * CPR, CPR)
                pltpu.sync_copy(acc_sh.at[rsl], acc_out.at[tb, c, rsl])
            return 0

        if TB == 1:
            per_tb(0, 0)
        else:
            lax.fori_loop(0, TB, per_tb, 0)

    return gat_edges


def _pad_h(h, dout):
    n = h.shape[0]
    return jnp.concatenate(
        [h, jnp.ones((n, 1), jnp.float32),
         jnp.zeros((n, WP - dout - 1), jnp.float32)], axis=1)


def _sd_of(h, a2):
    return lax.dot_general(a2, h, (((0,), (1,)), ((), ())),
                           preferred_element_type=jnp.float32,
                           precision=_PREC)


def _prep_body(act_ref, w_ref, a2_ref, hpad_ref, sd_ref):
    act = act_ref[0]
    h = jnp.dot(act, w_ref[...], preferred_element_type=jnp.float32,
                precision=_PREC)
    sd_ref[0] = _sd_of(h, a2_ref[...])
    hpad_ref[0] = _pad_h(h, H)


def _prep(act, w, a2, *, TB, din):
    return pl.pallas_call(
        _prep_body,
        grid=(TB,),
        in_specs=[
            pl.BlockSpec((1, N, din), lambda t: (t, 0, 0)),
            pl.BlockSpec((din, H), lambda t: (0, 0)),
            pl.BlockSpec((H, 2), lambda t: (0, 0)),
        ],
        out_specs=[
            pl.BlockSpec((1, N, WP), lambda t: (t, 0, 0)),
            pl.BlockSpec((1, 2, N), lambda t: (t, 0, 0)),
        ],
        out_shape=[
            jax.ShapeDtypeStruct((TB, N, WP), jnp.float32),
            jax.ShapeDtypeStruct((TB, 2, N), jnp.float32),
        ],
    )(act, w, a2)


def _act_of(acc_ref, b_ref):
    num = acc_ref[0, 0, :N] + acc_ref[0, 1, :N]
    den = num[:, H:H + 1] + 1e-16
    return jnp.maximum(num[:, :H] / den + b_ref[...], 0.0)


def _finprep_body(acc_ref, b_ref, w_ref, a2_ref, hpad_ref, sd_ref):
    act = _act_of(acc_ref, b_ref)
    h = jnp.dot(act, w_ref[...], preferred_element_type=jnp.float32,
                precision=_PREC)
    sd_ref[0] = _sd_of(h, a2_ref[...])
    hpad_ref[0] = _pad_h(h, H)


def _finprep(acc, b, w, a2, *, TB):
    return pl.pallas_call(
        _finprep_body,
        grid=(TB,),
        in_specs=[
            pl.BlockSpec((1, NC, NP, WP), lambda t: (t, 0, 0, 0)),
            pl.BlockSpec((1, H), lambda t: (0, 0)),
            pl.BlockSpec((H, H), lambda t: (0, 0)),
            pl.BlockSpec((H, 2), lambda t: (0, 0)),
        ],
        out_specs=[
            pl.BlockSpec((1, N, WP), lambda t: (t, 0, 0)),
            pl.BlockSpec((1, 2, N), lambda t: (t, 0, 0)),
        ],
        out_shape=[
            jax.ShapeDtypeStruct((TB, N, WP), jnp.float32),
            jax.ShapeDtypeStruct((TB, 2, N), jnp.float32),
        ],
    )(acc, b, w, a2)


def _finprep3_body(acc_ref, b_ref, w_ref, a2_ref, hpada_ref, hpadb_ref,
                   sd_ref):
    act = _act_of(acc_ref, b_ref)
    h = jnp.dot(act, w_ref[...], preferred_element_type=jnp.float32,
                precision=_PREC)
    sd_ref[0] = _sd_of(h, a2_ref[...])
    hpada_ref[0] = _pad_h(h[:, :H], H)
    hpadb_ref[0] = _pad_h(h[:, H:], H)


def _finprep3(acc, b, w, a2):
    return pl.pallas_call(
        _finprep3_body,
        grid=(1,),
        in_specs=[
            pl.BlockSpec((1, NC, NP, WP), lambda t: (t, 0, 0, 0)),
            pl.BlockSpec((1, H), lambda t: (0, 0)),
            pl.BlockSpec((H, F), lambda t: (0, 0)),
            pl.BlockSpec((F, 2), lambda t: (0, 0)),
        ],
        out_specs=[
            pl.BlockSpec((1, N, WP), lambda t: (t, 0, 0)),
            pl.BlockSpec((1, N, WP), lambda t: (t, 0, 0)),
            pl.BlockSpec((1, 2, N), lambda t: (t, 0, 0)),
        ],
        out_shape=[
            jax.ShapeDtypeStruct((1, N, WP), jnp.float32),
            jax.ShapeDtypeStruct((1, N, WP), jnp.float32),
            jax.ShapeDtypeStruct((1, 2, N), jnp.float32),
        ],
    )(acc, b, w, a2)


def _finpool_body(acc_ref, b_ref, pooled_ref):
    act = _act_of(acc_ref, b_ref)
    pooled_ref[0] = jnp.mean(act, axis=0, keepdims=True)


def _finpool(acc, b):
    return pl.pallas_call(
        _finpool_body,
        grid=(T,),
        in_specs=[
            pl.BlockSpec((1, NC, NP, WP), lambda t: (t, 0, 0, 0)),
            pl.BlockSpec((1, H), lambda t: (0, 0)),
        ],
        out_specs=pl.BlockSpec((1, 1, H), lambda t: (t, 0, 0)),
        out_shape=jax.ShapeDtypeStruct((T, 1, H), jnp.float32),
    )(acc, b)


def _lstm_body(pooled_ref, efw_ref, efb_ref, wih_ref, whh_ref, bih_ref,
               bhh_ref, hw_ref, hb_ref, agg_ref):
    zs = jnp.dot(pooled_ref[...], efw_ref[...],
                 preferred_element_type=jnp.float32,
                 precision=_PREC) + efb_ref[...]
    wih = wih_ref[...]
    whh = whh_ref[...]
    bsum = bih_ref[...] + bhh_ref[...]
    h = jnp.zeros((1, LH), jnp.float32)
    cst = jnp.zeros((1, LH), jnp.float32)
    for t in range(T):
        zt = zs[t:t + 1]
        gates = (lax.dot_general(zt, wih, (((1,), (1,)), ((), ())),
                                 precision=_PREC)
                 + lax.dot_general(h, whh, (((1,), (1,)), ((), ())),
                                   precision=_PREC) + bsum)
        i = jax.nn.sigmoid(gates[:, 0 * LH:1 * LH])
        f = jax.nn.sigmoid(gates[:, 1 * LH:2 * LH])
        g = jnp.tanh(gates[:, 2 * LH:3 * LH])
        o = jax.nn.sigmoid(gates[:, 3 * LH:4 * LH])
        cst = f * cst + i * g
        h = o * jnp.tanh(cst)
    agg_ref[...] = jnp.dot(h, hw_ref[...], preferred_element_type=jnp.float32,
                           precision=_PREC) + hb_ref[...]


def _lstm_head(pooled, efw, efb, wih, whh, bih, bhh, hw, hb):
    return pl.pallas_call(
        _lstm_body,
        out_shape=jax.ShapeDtypeStruct((1, L), jnp.float32),
    )(pooled, efw, efb, wih, whh, bih, bhh, hw, hb)


def _decfc_body(agg_ref, w_ref, b_ref, out_ref):
    out_ref[...] = jnp.maximum(
        jnp.dot(agg_ref[...], w_ref[...], preferred_element_type=jnp.float32,
                precision=_PREC) + b_ref[...], 0.0)


def _decfc(agg, w, b):
    BC = 32000
    G = (N * L) // BC
    return pl.pallas_call(
        _decfc_body,
        grid=(G,),
        in_specs=[
            pl.BlockSpec((1, L), lambda j: (0, 0)),
            pl.BlockSpec((L, BC), lambda j: (0, j)),
            pl.BlockSpec((1, BC), lambda j: (0, j)),
        ],
        out_specs=pl.BlockSpec((1, BC), lambda j: (0, j)),
        out_shape=jax.ShapeDtypeStruct((1, N * L), jnp.float32),
    )(agg, w, b)


def _final_body(acca_ref, accb_ref, b_ref, out_ref):
    numa = acca_ref[0, 0, :N] + acca_ref[0, 1, :N]
    dena = numa[:, H:H + 1] + 1e-16
    numb = accb_ref[0, 0, :N] + accb_ref[0, 1, :N]
    denb = numb[:, H:H + 1] + 1e-16
    out_ref[...] = jnp.concatenate(
        [numa[:, :H] / dena, numb[:, :H] / denb], axis=1) + b_ref[...]


def _final(acca, accb, b):
    return pl.pallas_call(
        _final_body,
        out_shape=jax.ShapeDtypeStruct((N, F), jnp.float32),
    )(acca, accb, b)


def _a2(p):
    return jnp.stack([p["a_src"], p["a_dst"]], axis=1)


def kernel(x, edge_index, params):
    esrc = edge_index[0]
    edst = edge_index[1]
    enc0, enc1 = params["enc"]
    dec = params["dec"]

    hpad, sd = _prep(x, enc0["W"], _a2(enc0), TB=T, din=F)
    acc = _make_gat_edges(T)(hpad.reshape(T * N, WP), sd, esrc, edst)

    hpad, sd = _finprep(acc, enc0["b"].reshape(1, H), enc1["W"], _a2(enc1),
                        TB=T)
    acc = _make_gat_edges(T)(hpad.reshape(T * N, WP), sd, esrc, edst)

    pooled = _finpool(acc, enc1["b"].reshape(1, H)).reshape(T, H)

    agg = _lstm_head(
        pooled, params["enc_fc_W"], params["enc_fc_b"].reshape(1, L),
        params["lstm_W_ih"], params["lstm_W_hh"],
        params["lstm_b_ih"].reshape(1, 4 * LH),
        params["lstm_b_hh"].reshape(1, 4 * LH),
        params["head_W"], params["head_b"].reshape(1, L))

    dec_in = _decfc(agg, params["dec_fc_W"],
                    params["dec_fc_b"].reshape(1, N * L)).reshape(1, N, L)

    hpad, sd = _prep(dec_in, dec[0]["W"], _a2(dec[0]), TB=1, din=L)
    acc = _make_gat_edges(1)(hpad.reshape(N, WP), sd, esrc, edst)

    hpad, sd = _finprep(acc, dec[0]["b"].reshape(1, H), dec[1]["W"],
                        _a2(dec[1]), TB=1)
    acc = _make_gat_edges(1)(hpad.reshape(N, WP), sd, esrc, edst)

    hpad, sd = _finprep(acc, dec[1]["b"].reshape(1, H), dec[2]["W"],
                        _a2(dec[2]), TB=1)
    acc = _make_gat_edges(1)(hpad.reshape(N, WP), sd, esrc, edst)

    hpada, hpadb, sd = _finprep3(acc, dec[2]["b"].reshape(1, H), dec[3]["W"],
                                 _a2(dec[3]))
    acca = _make_gat_edges(1)(hpada.reshape(N, WP), sd, esrc, edst)
    accb = _make_gat_edges(1)(hpadb.reshape(N, WP), sd, esrc, edst)

    return _final(acca, accb, dec[3]["b"].reshape(1, F))


# speedup vs baseline: 23.2518x; 1.1124x over previous
import functools

import jax
import jax.numpy as jnp
from jax import lax
from jax.experimental import pallas as pl
from jax.experimental.pallas import tpu as pltpu
from jax.experimental.pallas import tpu_sc as plsc

T, N, F, E = 8, 10000, 128, 320000
H, L, LH = 64, 32, 64

NC, NS = 2, 16
NW = NC * NS
EW = E // NW
CH = 80
NCHUNK = EW // CH
NP = 10240
ROWS_PER_TILE = NP // NS
CPR = 128
NCP = ROWS_PER_TILE // CPR
ZPR = 32
NZP = ROWS_PER_TILE // ZPR
SEG = 2000
NSEG = EW // SEG
CPS = SEG // CH
WP = 80
NV_SCALE = 5

_PREC = lax.Precision.HIGHEST


@functools.lru_cache(maxsize=None)
def _make_gat_edges(TB: int):
    mesh = plsc.VectorSubcoreMesh(core_axis_name="c", subcore_axis_name="s")

    @functools.partial(
        pl.kernel,
        out_type=jax.ShapeDtypeStruct((TB, NC, NP, WP), jnp.float32),
        mesh=mesh,
        compiler_params=pltpu.CompilerParams(
            needs_layout_passes=False, use_tc_tiling_on_sc=False),
        scratch_types=[
            pltpu.VMEM_SHARED((NP, WP), jnp.float32),
            pltpu.VMEM((N,), jnp.float32),
            pltpu.VMEM((N,), jnp.float32),
            pltpu.VMEM((SEG,), jnp.int32),
            pltpu.VMEM((SEG,), jnp.int32),
            pltpu.VMEM((CH,), jnp.int32),
            pltpu.VMEM((CH,), jnp.int32),
            pltpu.VMEM((CH, WP), jnp.float32),
            pltpu.VMEM((ZPR, WP), jnp.float32),
            pltpu.SemaphoreType.DMA,
        ],
    )
    def gat_edges(hpad_hbm, sd_hbm, esrc_hbm, edst_hbm, acc_out,
                  acc_sh, s_v, d_v, srcs_v, dsts_v, srci_v, dsti_v,
                  rows_v, zrows_v, sem):
        c = lax.axis_index("c")
        s = lax.axis_index("s")
        wid = c * NS + s
        row0 = s * ROWS_PER_TILE

        def _zr(r, _):
            for v in range(WP // 16):
                zrows_v[r, pl.ds(v * 16, 16)] = jnp.zeros((16,), jnp.float32)
            return 0
        lax.fori_loop(0, ZPR, _zr, 0)

        def per_tb(tb, _):
            for z in range(NZP):
                pltpu.sync_copy(
                    zrows_v, acc_sh.at[pl.ds(row0 + z * ZPR, ZPR)])
            pltpu.sync_copy(sd_hbm.at[tb, 0], s_v)
            pltpu.sync_copy(sd_hbm.at[tb, 1], d_v)
            plsc.subcore_barrier()

            base = tb * N

            def strip(g, _):
                e0 = wid * EW + g * SEG
                pltpu.sync_copy(esrc_hbm.at[pl.ds(e0, SEG)], srcs_v)
                pltpu.sync_copy(edst_hbm.at[pl.ds(e0, SEG)], dsts_v)

                def p2(ci, _):
                    exs = []
                    for w in range(CH // 16):
                        esl = pl.ds(ci * CH + w * 16, 16)
                        srcv = srcs_v[esl]
                        dstv = dsts_v[esl]
                        sv = plsc.load_gather(s_v, [srcv])
                        dv = plsc.load_gather(d_v, [dstv])
                        logit = sv + dv
                        e = jnp.maximum(logit, 0.2 * logit)
                        exs.append(jnp.exp(e))
                        srci_v[pl.ds(w * 16, 16)] = srcv + base
                        dsti_v[pl.ds(w * 16, 16)] = dstv
                    pltpu.async_copy(
                        hpad_hbm.at[srci_v], rows_v, sem).wait()
                    for w in range(CH // 16):
                        exv = exs[w]
                        for k2 in range(16):
                            exb = jnp.full((16,), exv[k2], jnp.float32)
                            row = w * 16 + k2
                            for v in range(NV_SCALE):
                                rows_v[row, pl.ds(v * 16, 16)] = (
                                    rows_v[row, pl.ds(v * 16, 16)] * exb)
                    pltpu.sync_copy(rows_v, acc_sh.at[dsti_v], add=True)
                    return 0
                lax.fori_loop(0, CPS, p2, 0)
                return 0
            lax.fori_loop(0, NSEG, strip, 0)
            plsc.subcore_barrier()

            for z in range(NCP):
                rsl = pl.ds(row0 + z * CPR, CPR)
                pltpu.sync_copy(acc_sh.at[rsl], acc_out.at[tb, c, rsl])
            return 0

        if TB == 1:
            per_tb(0, 0)
        else:
            lax.fori_loop(0, TB, per_tb, 0)

    return gat_edges


def _pad_h(h, dout):
    n = h.shape[0]
    return jnp.concatenate(
        [h, jnp.ones((n, 1), jnp.float32),
         jnp.zeros((n, WP - dout - 1), jnp.float32)], axis=1)


def _sd_of(h, a2):
    return lax.dot_general(a2, h, (((0,), (1,)), ((), ())),
                           preferred_element_type=jnp.float32,
                           precision=_PREC)


def _prep_body(act_ref, w_ref, a2_ref, hpad_ref, sd_ref):
    act = act_ref[0]
    h = jnp.dot(act, w_ref[...], preferred_element_type=jnp.float32,
                precision=_PREC)
    sd_ref[0] = _sd_of(h, a2_ref[...])
    hpad_ref[0] = _pad_h(h, H)


def _prep(act, w, a2, *, TB, din):
    return pl.pallas_call(
        _prep_body,
        grid=(TB,),
        in_specs=[
            pl.BlockSpec((1, N, din), lambda t: (t, 0, 0)),
            pl.BlockSpec((din, H), lambda t: (0, 0)),
            pl.BlockSpec((H, 2), lambda t: (0, 0)),
        ],
        out_specs=[
            pl.BlockSpec((1, N, WP), lambda t: (t, 0, 0)),
            pl.BlockSpec((1, 2, N), lambda t: (t, 0, 0)),
        ],
        out_shape=[
            jax.ShapeDtypeStruct((TB, N, WP), jnp.float32),
            jax.ShapeDtypeStruct((TB, 2, N), jnp.float32),
        ],
    )(act, w, a2)


def _act_of(acc_ref, b_ref):
    num = acc_ref[0, 0, :N] + acc_ref[0, 1, :N]
    den = num[:, H:H + 1] + 1e-16
    return jnp.maximum(num[:, :H] / den + b_ref[...], 0.0)


def _finprep_body(acc_ref, b_ref, w_ref, a2_ref, hpad_ref, sd_ref):
    act = _act_of(acc_ref, b_ref)
    h = jnp.dot(act, w_ref[...], preferred_element_type=jnp.float32,
                precision=_PREC)
    sd_ref[0] = _sd_of(h, a2_ref[...])
    hpad_ref[0] = _pad_h(h, H)


def _finprep(acc, b, w, a2, *, TB):
    return pl.pallas_call(
        _finprep_body,
        grid=(TB,),
        in_specs=[
            pl.BlockSpec((1, NC, NP, WP), lambda t: (t, 0, 0, 0)),
            pl.BlockSpec((1, H), lambda t: (0, 0)),
            pl.BlockSpec((H, H), lambda t: (0, 0)),
            pl.BlockSpec((H, 2), lambda t: (0, 0)),
        ],
        out_specs=[
            pl.BlockSpec((1, N, WP), lambda t: (t, 0, 0)),
            pl.BlockSpec((1, 2, N), lambda t: (t, 0, 0)),
        ],
        out_shape=[
            jax.ShapeDtypeStruct((TB, N, WP), jnp.float32),
            jax.ShapeDtypeStruct((TB, 2, N), jnp.float32),
        ],
    )(acc, b, w, a2)


def _finprep3_body(acc_ref, b_ref, w_ref, a2_ref, hpada_ref, hpadb_ref,
                   sd_ref):
    act = _act_of(acc_ref, b_ref)
    h = jnp.dot(act, w_ref[...], preferred_element_type=jnp.float32,
                precision=_PREC)
    sd_ref[0] = _sd_of(h, a2_ref[...])
    hpada_ref[0] = _pad_h(h[:, :H], H)
    hpadb_ref[0] = _pad_h(h[:, H:], H)


def _finprep3(acc, b, w, a2):
    return pl.pallas_call(
        _finprep3_body,
        grid=(1,),
        in_specs=[
            pl.BlockSpec((1, NC, NP, WP), lambda t: (t, 0, 0, 0)),
            pl.BlockSpec((1, H), lambda t: (0, 0)),
            pl.BlockSpec((H, F), lambda t: (0, 0)),
            pl.BlockSpec((F, 2), lambda t: (0, 0)),
        ],
        out_specs=[
            pl.BlockSpec((1, N, WP), lambda t: (t, 0, 0)),
            pl.BlockSpec((1, N, WP), lambda t: (t, 0, 0)),
            pl.BlockSpec((1, 2, N), lambda t: (t, 0, 0)),
        ],
        out_shape=[
            jax.ShapeDtypeStruct((1, N, WP), jnp.float32),
            jax.ShapeDtypeStruct((1, N, WP), jnp.float32),
            jax.ShapeDtypeStruct((1, 2, N), jnp.float32),
        ],
    )(acc, b, w, a2)


def _finpool_body(acc_ref, b_ref, pooled_ref):
    act = _act_of(acc_ref, b_ref)
    pooled_ref[0] = jnp.mean(act, axis=0, keepdims=True)


def _finpool(acc, b):
    return pl.pallas_call(
        _finpool_body,
        grid=(T,),
        in_specs=[
            pl.BlockSpec((1, NC, NP, WP), lambda t: (t, 0, 0, 0)),
            pl.BlockSpec((1, H), lambda t: (0, 0)),
        ],
        out_specs=pl.BlockSpec((1, 1, H), lambda t: (t, 0, 0)),
        out_shape=jax.ShapeDtypeStruct((T, 1, H), jnp.float32),
    )(acc, b)


def _lstm_body(pooled_ref, efw_ref, efb_ref, wih_ref, whh_ref, bih_ref,
               bhh_ref, hw_ref, hb_ref, agg_ref):
    zs = jnp.dot(pooled_ref[...], efw_ref[...],
                 preferred_element_type=jnp.float32,
                 precision=_PREC) + efb_ref[...]
    wih = wih_ref[...]
    whh = whh_ref[...]
    bsum = bih_ref[...] + bhh_ref[...]
    h = jnp.zeros((1, LH), jnp.float32)
    cst = jnp.zeros((1, LH), jnp.float32)
    for t in range(T):
        zt = zs[t:t + 1]
        gates = (lax.dot_general(zt, wih, (((1,), (1,)), ((), ())),
                                 precision=_PREC)
                 + lax.dot_general(h, whh, (((1,), (1,)), ((), ())),
                                   precision=_PREC) + bsum)
        i = jax.nn.sigmoid(gates[:, 0 * LH:1 * LH])
        f = jax.nn.sigmoid(gates[:, 1 * LH:2 * LH])
        g = jnp.tanh(gates[:, 2 * LH:3 * LH])
        o = jax.nn.sigmoid(gates[:, 3 * LH:4 * LH])
        cst = f * cst + i * g
        h = o * jnp.tanh(cst)
    agg_ref[...] = jnp.dot(h, hw_ref[...], preferred_element_type=jnp.float32,
                           precision=_PREC) + hb_ref[...]


def _lstm_head(pooled, efw, efb, wih, whh, bih, bhh, hw, hb):
    return pl.pallas_call(
        _lstm_body,
        out_shape=jax.ShapeDtypeStruct((1, L), jnp.float32),
    )(pooled, efw, efb, wih, whh, bih, bhh, hw, hb)


def _decfc_body(agg_ref, w_ref, b_ref, out_ref):
    out_ref[...] = jnp.maximum(
        jnp.dot(agg_ref[...], w_ref[...], preferred_element_type=jnp.float32,
                precision=_PREC) + b_ref[...], 0.0)


def _decfc(agg, w, b):
    BC = 32000
    G = (N * L) // BC
    return pl.pallas_call(
        _decfc_body,
        grid=(G,),
        in_specs=[
            pl.BlockSpec((1, L), lambda j: (0, 0)),
            pl.BlockSpec((L, BC), lambda j: (0, j)),
            pl.BlockSpec((1, BC), lambda j: (0, j)),
        ],
        out_specs=pl.BlockSpec((1, BC), lambda j: (0, j)),
        out_shape=jax.ShapeDtypeStruct((1, N * L), jnp.float32),
    )(agg, w, b)


def _final_body(acca_ref, accb_ref, b_ref, out_ref):
    numa = acca_ref[0, 0, :N] + acca_ref[0, 1, :N]
    dena = numa[:, H:H + 1] + 1e-16
    numb = accb_ref[0, 0, :N] + accb_ref[0, 1, :N]
    denb = numb[:, H:H + 1] + 1e-16
    out_ref[...] = jnp.concatenate(
        [numa[:, :H] / dena, numb[:, :H] / denb], axis=1) + b_ref[...]


def _final(acca, accb, b):
    return pl.pallas_call(
        _final_body,
        out_shape=jax.ShapeDtypeStruct((N, F), jnp.float32),
    )(acca, accb, b)


def _a2(p):
    return jnp.stack([p["a_src"], p["a_dst"]], axis=1)


def kernel(x, edge_index, params):
    esrc = edge_index[0]
    edst = edge_index[1]
    enc0, enc1 = params["enc"]
    dec = params["dec"]

    hpad, sd = _prep(x, enc0["W"], _a2(enc0), TB=T, din=F)
    acc = _make_gat_edges(T)(hpad.reshape(T * N, WP), sd, esrc, edst)

    hpad, sd = _finprep(acc, enc0["b"].reshape(1, H), enc1["W"], _a2(enc1),
                        TB=T)
    acc = _make_gat_edges(T)(hpad.reshape(T * N, WP), sd, esrc, edst)

    pooled = _finpool(acc, enc1["b"].reshape(1, H)).reshape(T, H)

    agg = _lstm_head(
        pooled, params["enc_fc_W"], params["enc_fc_b"].reshape(1, L),
        params["lstm_W_ih"], params["lstm_W_hh"],
        params["lstm_b_ih"].reshape(1, 4 * LH),
        params["lstm_b_hh"].reshape(1, 4 * LH),
        params["head_W"], params["head_b"].reshape(1, L))

    dec_in = _decfc(agg, params["dec_fc_W"],
                    params["dec_fc_b"].reshape(1, N * L)).reshape(1, N, L)

    hpad, sd = _prep(dec_in, dec[0]["W"], _a2(dec[0]), TB=1, din=L)
    acc = _make_gat_edges(1)(hpad.reshape(N, WP), sd, esrc, edst)

    hpad, sd = _finprep(acc, dec[0]["b"].reshape(1, H), dec[1]["W"],
                        _a2(dec[1]), TB=1)
    acc = _make_gat_edges(1)(hpad.reshape(N, WP), sd, esrc, edst)

    hpad, sd = _finprep(acc, dec[1]["b"].reshape(1, H), dec[2]["W"],
                        _a2(dec[2]), TB=1)
    acc = _make_gat_edges(1)(hpad.reshape(N, WP), sd, esrc, edst)

    hpada, hpadb, sd = _finprep3(acc, dec[2]["b"].reshape(1, H), dec[3]["W"],
                                 _a2(dec[3]))
    acca = _make_gat_edges(1)(hpada.reshape(N, WP), sd, esrc, edst)
    accb = _make_gat_edges(1)(hpadb.reshape(N, WP), sd, esrc, edst)

    return _final(acca, accb, dec[3]["b"].reshape(1, F))


# speedup vs baseline: 34.0290x; 1.4635x over previous
import functools

import jax
import jax.numpy as jnp
from jax import lax
from jax.experimental import pallas as pl
from jax.experimental.pallas import tpu as pltpu
from jax.experimental.pallas import tpu_sc as plsc

T, N, F, E = 8, 10000, 128, 320000
H, L, LH = 64, 32, 64

NC, NS = 2, 16
NW = NC * NS
EW = E // NW
CH = 80
NCHUNK = EW // CH
NP = 10240
ROWS_PER_TILE = NP // NS
CPR = 128
NCP = ROWS_PER_TILE // CPR
ZPR = 32
NZP = ROWS_PER_TILE // ZPR
SEG = 2000
NSEG = EW // SEG
CPS = SEG // CH
WP = 80
NV_SCALE = 5

_PREC = lax.Precision.HIGHEST


@functools.lru_cache(maxsize=None)
def _make_gat_edges(TB: int):
    mesh = plsc.VectorSubcoreMesh(core_axis_name="c", subcore_axis_name="s")

    @functools.partial(
        pl.kernel,
        out_type=jax.ShapeDtypeStruct((TB, NC, NP, WP), jnp.float32),
        mesh=mesh,
        compiler_params=pltpu.CompilerParams(
            needs_layout_passes=False, use_tc_tiling_on_sc=False),
        scratch_types=[
            pltpu.VMEM_SHARED((NP, WP), jnp.float32),
            pltpu.VMEM((N,), jnp.float32),
            pltpu.VMEM((N,), jnp.float32),
            pltpu.VMEM((SEG,), jnp.int32),
            pltpu.VMEM((SEG,), jnp.int32),
            pltpu.VMEM((SEG,), jnp.float32),
            pltpu.VMEM((CH,), jnp.int32),
            pltpu.VMEM((CH,), jnp.int32),
            pltpu.VMEM((CH,), jnp.int32),
            pltpu.VMEM((CH,), jnp.int32),
            pltpu.VMEM((CH, WP), jnp.float32),
            pltpu.VMEM((CH, WP), jnp.float32),
            pltpu.VMEM((ZPR, WP), jnp.float32),
            pltpu.SemaphoreType.DMA,
            pltpu.SemaphoreType.DMA,
            pltpu.SemaphoreType.DMA,
            pltpu.SemaphoreType.DMA,
        ],
    )
    def gat_edges(hpad_hbm, sd_hbm, esrc_hbm, edst_hbm, acc_out,
                  acc_sh, s_v, d_v, srcs_v, dsts_v, exs_v,
                  srci0_v, dsti0_v, srci1_v, dsti1_v,
                  rows0_v, rows1_v, zrows_v, sg0, sg1, ss0, ss1):
        c = lax.axis_index("c")
        s = lax.axis_index("s")
        wid = c * NS + s
        row0 = s * ROWS_PER_TILE

        def _zr(r, _):
            for v in range(WP // 16):
                zrows_v[r, pl.ds(v * 16, 16)] = jnp.zeros((16,), jnp.float32)
            return 0
        lax.fori_loop(0, ZPR, _zr, 0)

        def per_tb(tb, _):
            for z in range(NZP):
                pltpu.sync_copy(
                    zrows_v, acc_sh.at[pl.ds(row0 + z * ZPR, ZPR)])
            pltpu.sync_copy(sd_hbm.at[tb, 0], s_v)
            pltpu.sync_copy(sd_hbm.at[tb, 1], d_v)
            plsc.subcore_barrier()

            base = tb * N
            bufs = ((srci0_v, dsti0_v, rows0_v, sg0, ss0),
                    (srci1_v, dsti1_v, rows1_v, sg1, ss1))

            def scale(ci, rows_v):
                def sc_w(w, _):
                    exv = exs_v[pl.ds(ci * CH + w * 16, 16)]
                    for k2 in range(16):
                        exb = jnp.full((16,), exv[k2], jnp.float32)
                        row = w * 16 + k2
                        for v in range(NV_SCALE):
                            rows_v[row, pl.ds(v * 16, 16)] = (
                                rows_v[row, pl.ds(v * 16, 16)] * exb)
                    return 0
                lax.fori_loop(0, CH // 16, sc_w, 0)

            def step(ci, b):
                srci_b, dsti_b, rows_b, sg_b, ss_b = bufs[b]
                srci_n, dsti_n, rows_n, sg_n, ss_n = bufs[1 - b]

                def wait_sct():
                    pltpu.make_async_copy(
                        rows_b, acc_sh.at[dsti_b], ss_b).wait()
                if isinstance(ci, int):
                    if ci >= 2:
                        wait_sct()
                else:
                    pl.when(ci >= 2)(wait_sct)

                for w in range(CH // 16):
                    esl = pl.ds(ci * CH + w * 16, 16)
                    srci_b[pl.ds(w * 16, 16)] = srcs_v[esl] + base
                    dsti_b[pl.ds(w * 16, 16)] = dsts_v[esl]
                pltpu.async_copy(hpad_hbm.at[srci_b], rows_b, sg_b)

                def fin_prev():
                    pltpu.make_async_copy(
                        hpad_hbm.at[srci_n], rows_n, sg_n).wait()
                    scale(ci - 1, rows_n)
                    pltpu.async_copy(
                        rows_n, acc_sh.at[dsti_n], ss_n, add=True)
                if isinstance(ci, int):
                    if ci >= 1:
                        fin_prev()
                else:
                    pl.when(ci >= 1)(fin_prev)

            def strip(g, _):
                e0 = wid * EW + g * SEG
                pltpu.sync_copy(esrc_hbm.at[pl.ds(e0, SEG)], srcs_v)
                pltpu.sync_copy(edst_hbm.at[pl.ds(e0, SEG)], dsts_v)

                def p1(j, _):
                    sl = pl.ds(j * 16, 16)
                    sv = plsc.load_gather(s_v, [srcs_v[sl]])
                    dv = plsc.load_gather(d_v, [dsts_v[sl]])
                    logit = sv + dv
                    e = jnp.maximum(logit, 0.2 * logit)
                    exs_v[sl] = jnp.exp(e)
                    return 0
                lax.fori_loop(0, SEG // 16, p1, 0)

                def pair(j, _):
                    step(2 * j, 0)
                    step(2 * j + 1, 1)
                    return 0
                lax.fori_loop(0, (CPS - 1) // 2, pair, 0)
                step(CPS - 1, 0)
                pltpu.make_async_copy(
                    hpad_hbm.at[srci0_v], rows0_v, sg0).wait()
                scale(CPS - 1, rows0_v)
                pltpu.async_copy(rows0_v, acc_sh.at[dsti0_v], ss0, add=True)
                pltpu.make_async_copy(
                    rows1_v, acc_sh.at[dsti1_v], ss1).wait()
                pltpu.make_async_copy(
                    rows0_v, acc_sh.at[dsti0_v], ss0).wait()
                return 0
            lax.fori_loop(0, NSEG, strip, 0)
            plsc.subcore_barrier()

            for z in range(NCP):
                rsl = pl.ds(row0 + z * CPR, CPR)
                pltpu.sync_copy(acc_sh.at[rsl], acc_out.at[tb, c, rsl])
            return 0

        if TB == 1:
            per_tb(0, 0)
        else:
            lax.fori_loop(0, TB, per_tb, 0)

    return gat_edges


def _pad_h(h, dout):
    n = h.shape[0]
    return jnp.concatenate(
        [h, jnp.ones((n, 1), jnp.float32),
         jnp.zeros((n, WP - dout - 1), jnp.float32)], axis=1)


def _sd_of(h, a2):
    return lax.dot_general(a2, h, (((0,), (1,)), ((), ())),
                           preferred_element_type=jnp.float32,
                           precision=_PREC)


def _prep_body(act_ref, w_ref, a2_ref, hpad_ref, sd_ref):
    act = act_ref[0]
    h = jnp.dot(act, w_ref[...], preferred_element_type=jnp.float32,
                precision=_PREC)
    sd_ref[0] = _sd_of(h, a2_ref[...])
    hpad_ref[0] = _pad_h(h, H)


def _prep(act, w, a2, *, TB, din):
    return pl.pallas_call(
        _prep_body,
        grid=(TB,),
        in_specs=[
            pl.BlockSpec((1, N, din), lambda t: (t, 0, 0)),
            pl.BlockSpec((din, H), lambda t: (0, 0)),
            pl.BlockSpec((H, 2), lambda t: (0, 0)),
        ],
        out_specs=[
            pl.BlockSpec((1, N, WP), lambda t: (t, 0, 0)),
            pl.BlockSpec((1, 2, N), lambda t: (t, 0, 0)),
        ],
        out_shape=[
            jax.ShapeDtypeStruct((TB, N, WP), jnp.float32),
            jax.ShapeDtypeStruct((TB, 2, N), jnp.float32),
        ],
    )(act, w, a2)


def _act_of(acc_ref, b_ref):
    num = acc_ref[0, 0, :N] + acc_ref[0, 1, :N]
    den = num[:, H:H + 1] + 1e-16
    return jnp.maximum(num[:, :H] / den + b_ref[...], 0.0)


def _finprep_body(acc_ref, b_ref, w_ref, a2_ref, hpad_ref, sd_ref):
    act = _act_of(acc_ref, b_ref)
    h = jnp.dot(act, w_ref[...], preferred_element_type=jnp.float32,
                precision=_PREC)
    sd_ref[0] = _sd_of(h, a2_ref[...])
    hpad_ref[0] = _pad_h(h, H)


def _finprep(acc, b, w, a2, *, TB):
    return pl.pallas_call(
        _finprep_body,
        grid=(TB,),
        in_specs=[
            pl.BlockSpec((1, NC, NP, WP), lambda t: (t, 0, 0, 0)),
            pl.BlockSpec((1, H), lambda t: (0, 0)),
            pl.BlockSpec((H, H), lambda t: (0, 0)),
            pl.BlockSpec((H, 2), lambda t: (0, 0)),
        ],
        out_specs=[
            pl.BlockSpec((1, N, WP), lambda t: (t, 0, 0)),
            pl.BlockSpec((1, 2, N), lambda t: (t, 0, 0)),
        ],
        out_shape=[
            jax.ShapeDtypeStruct((TB, N, WP), jnp.float32),
            jax.ShapeDtypeStruct((TB, 2, N), jnp.float32),
        ],
    )(acc, b, w, a2)


def _finprep3_body(acc_ref, b_ref, w_ref, a2_ref, hpada_ref, hpadb_ref,
                   sd_ref):
    act = _act_of(acc_ref, b_ref)
    h = jnp.dot(act, w_ref[...], preferred_element_type=jnp.float32,
                precision=_PREC)
    sd_ref[0] = _sd_of(h, a2_ref[...])
    hpada_ref[0] = _pad_h(h[:, :H], H)
    hpadb_ref[0] = _pad_h(h[:, H:], H)


def _finprep3(acc, b, w, a2):
    return pl.pallas_call(
        _finprep3_body,
        grid=(1,),
        in_specs=[
            pl.BlockSpec((1, NC, NP, WP), lambda t: (t, 0, 0, 0)),
            pl.BlockSpec((1, H), lambda t: (0, 0)),
            pl.BlockSpec((H, F), lambda t: (0, 0)),
            pl.BlockSpec((F, 2), lambda t: (0, 0)),
        ],
        out_specs=[
            pl.BlockSpec((1, N, WP), lambda t: (t, 0, 0)),
            pl.BlockSpec((1, N, WP), lambda t: (t, 0, 0)),
            pl.BlockSpec((1, 2, N), lambda t: (t, 0, 0)),
        ],
        out_shape=[
            jax.ShapeDtypeStruct((1, N, WP), jnp.float32),
            jax.ShapeDtypeStruct((1, N, WP), jnp.float32),
            jax.ShapeDtypeStruct((1, 2, N), jnp.float32),
        ],
    )(acc, b, w, a2)


def _finpool_body(acc_ref, b_ref, pooled_ref):
    act = _act_of(acc_ref, b_ref)
    pooled_ref[0] = jnp.mean(act, axis=0, keepdims=True)


def _finpool(acc, b):
    return pl.pallas_call(
        _finpool_body,
        grid=(T,),
        in_specs=[
            pl.BlockSpec((1, NC, NP, WP), lambda t: (t, 0, 0, 0)),
            pl.BlockSpec((1, H), lambda t: (0, 0)),
        ],
        out_specs=pl.BlockSpec((1, 1, H), lambda t: (t, 0, 0)),
        out_shape=jax.ShapeDtypeStruct((T, 1, H), jnp.float32),
    )(acc, b)


def _lstm_body(pooled_ref, efw_ref, efb_ref, wih_ref, whh_ref, bih_ref,
               bhh_ref, hw_ref, hb_ref, agg_ref):
    zs = jnp.dot(pooled_ref[...], efw_ref[...],
                 preferred_element_type=jnp.float32,
                 precision=_PREC) + efb_ref[...]
    wih = wih_ref[...]
    whh = whh_ref[...]
    bsum = bih_ref[...] + bhh_ref[...]
    h = jnp.zeros((1, LH), jnp.float32)
    cst = jnp.zeros((1, LH), jnp.float32)
    for t in range(T):
        zt = zs[t:t + 1]
        gates = (lax.dot_general(zt, wih, (((1,), (1,)), ((), ())),
                                 precision=_PREC)
                 + lax.dot_general(h, whh, (((1,), (1,)), ((), ())),
                                   precision=_PREC) + bsum)
        i = jax.nn.sigmoid(gates[:, 0 * LH:1 * LH])
        f = jax.nn.sigmoid(gates[:, 1 * LH:2 * LH])
        g = jnp.tanh(gates[:, 2 * LH:3 * LH])
        o = jax.nn.sigmoid(gates[:, 3 * LH:4 * LH])
        cst = f * cst + i * g
        h = o * jnp.tanh(cst)
    agg_ref[...] = jnp.dot(h, hw_ref[...], preferred_element_type=jnp.float32,
                           precision=_PREC) + hb_ref[...]


def _lstm_head(pooled, efw, efb, wih, whh, bih, bhh, hw, hb):
    return pl.pallas_call(
        _lstm_body,
        out_shape=jax.ShapeDtypeStruct((1, L), jnp.float32),
    )(pooled, efw, efb, wih, whh, bih, bhh, hw, hb)


def _decfc_body(agg_ref, w_ref, b_ref, out_ref):
    out_ref[...] = jnp.maximum(
        jnp.dot(agg_ref[...], w_ref[...], preferred_element_type=jnp.float32,
                precision=_PREC) + b_ref[...], 0.0)


def _decfc(agg, w, b):
    BC = 32000
    G = (N * L) // BC
    return pl.pallas_call(
        _decfc_body,
        grid=(G,),
        in_specs=[
            pl.BlockSpec((1, L), lambda j: (0, 0)),
            pl.BlockSpec((L, BC), lambda j: (0, j)),
            pl.BlockSpec((1, BC), lambda j: (0, j)),
        ],
        out_specs=pl.BlockSpec((1, BC), lambda j: (0, j)),
        out_shape=jax.ShapeDtypeStruct((1, N * L), jnp.float32),
    )(agg, w, b)


def _final_body(acca_ref, accb_ref, b_ref, out_ref):
    numa = acca_ref[0, 0, :N] + acca_ref[0, 1, :N]
    dena = numa[:, H:H + 1] + 1e-16
    numb = accb_ref[0, 0, :N] + accb_ref[0, 1, :N]
    denb = numb[:, H:H + 1] + 1e-16
    out_ref[...] = jnp.concatenate(
        [numa[:, :H] / dena, numb[:, :H] / denb], axis=1) + b_ref[...]


def _final(acca, accb, b):
    return pl.pallas_call(
        _final_body,
        out_shape=jax.ShapeDtypeStruct((N, F), jnp.float32),
    )(acca, accb, b)


def _a2(p):
    return jnp.stack([p["a_src"], p["a_dst"]], axis=1)


def kernel(x, edge_index, params):
    esrc = edge_index[0]
    edst = edge_index[1]
    enc0, enc1 = params["enc"]
    dec = params["dec"]

    hpad, sd = _prep(x, enc0["W"], _a2(enc0), TB=T, din=F)
    acc = _make_gat_edges(T)(hpad.reshape(T * N, WP), sd, esrc, edst)

    hpad, sd = _finprep(acc, enc0["b"].reshape(1, H), enc1["W"], _a2(enc1),
                        TB=T)
    acc = _make_gat_edges(T)(hpad.reshape(T * N, WP), sd, esrc, edst)

    pooled = _finpool(acc, enc1["b"].reshape(1, H)).reshape(T, H)

    agg = _lstm_head(
        pooled, params["enc_fc_W"], params["enc_fc_b"].reshape(1, L),
        params["lstm_W_ih"], params["lstm_W_hh"],
        params["lstm_b_ih"].reshape(1, 4 * LH),
        params["lstm_b_hh"].reshape(1, 4 * LH),
        params["head_W"], params["head_b"].reshape(1, L))

    dec_in = _decfc(agg, params["dec_fc_W"],
                    params["dec_fc_b"].reshape(1, N * L)).reshape(1, N, L)

    hpad, sd = _prep(dec_in, dec[0]["W"], _a2(dec[0]), TB=1, din=L)
    acc = _make_gat_edges(1)(hpad.reshape(N, WP), sd, esrc, edst)

    hpad, sd = _finprep(acc, dec[0]["b"].reshape(1, H), dec[1]["W"],
                        _a2(dec[1]), TB=1)
    acc = _make_gat_edges(1)(hpad.reshape(N, WP), sd, esrc, edst)

    hpad, sd = _finprep(acc, dec[1]["b"].reshape(1, H), dec[2]["W"],
                        _a2(dec[2]), TB=1)
    acc = _make_gat_edges(1)(hpad.reshape(N, WP), sd, esrc, edst)

    hpada, hpadb, sd = _finprep3(acc, dec[2]["b"].reshape(1, H), dec[3]["W"],
                                 _a2(dec[3]))
    acca = _make_gat_edges(1)(hpada.reshape(N, WP), sd, esrc, edst)
    accb = _make_gat_edges(1)(hpadb.reshape(N, WP), sd, esrc, edst)

    return _final(acca, accb, dec[3]["b"].reshape(1, F))


# speedup vs baseline: 34.8449x; 1.0240x over previous
import functools

import jax
import jax.numpy as jnp
from jax import lax
from jax.experimental import pallas as pl
from jax.experimental.pallas import tpu as pltpu
from jax.experimental.pallas import tpu_sc as plsc

T, N, F, E = 8, 10000, 128, 320000
H, L, LH = 64, 32, 64

NC, NS = 2, 16
NW = NC * NS
EW = E // NW
CH = 80
NCHUNK = EW // CH
NP = 10240
ROWS_PER_TILE = NP // NS
CPR = 128
NCP = ROWS_PER_TILE // CPR
ZPR = 128
NZP = ROWS_PER_TILE // ZPR
SEG = 2000
NSEG = EW // SEG
CPS = SEG // CH
WP = 80
NV_SCALE = 5

_PREC = lax.Precision.HIGHEST


@functools.lru_cache(maxsize=None)
def _make_gat_edges(TB: int):
    mesh = plsc.VectorSubcoreMesh(core_axis_name="c", subcore_axis_name="s")

    @functools.partial(
        pl.kernel,
        out_type=jax.ShapeDtypeStruct((TB, NC, NP, WP), jnp.float32),
        mesh=mesh,
        compiler_params=pltpu.CompilerParams(
            needs_layout_passes=False, use_tc_tiling_on_sc=False),
        scratch_types=[
            pltpu.VMEM_SHARED((NP, WP), jnp.float32),
            pltpu.VMEM((N,), jnp.float32),
            pltpu.VMEM((N,), jnp.float32),
            pltpu.VMEM((SEG,), jnp.int32),
            pltpu.VMEM((SEG,), jnp.int32),
            pltpu.VMEM((SEG,), jnp.float32),
            pltpu.VMEM((CH,), jnp.int32),
            pltpu.VMEM((CH,), jnp.int32),
            pltpu.VMEM((CH,), jnp.int32),
            pltpu.VMEM((CH,), jnp.int32),
            pltpu.VMEM((CH, WP), jnp.float32),
            pltpu.VMEM((CH, WP), jnp.float32),
            pltpu.VMEM((ZPR, WP), jnp.float32),
            pltpu.SemaphoreType.DMA,
            pltpu.SemaphoreType.DMA,
            pltpu.SemaphoreType.DMA,
            pltpu.SemaphoreType.DMA,
            pltpu.SemaphoreType.DMA,
        ],
    )
    def gat_edges(hpad_hbm, sd_hbm, esrc_hbm, edst_hbm, acc_out,
                  acc_sh, s_v, d_v, srcs_v, dsts_v, exs_v,
                  srci0_v, dsti0_v, srci1_v, dsti1_v,
                  rows0_v, rows1_v, zrows_v, sg0, sg1, ss0, ss1, sm):
        c = lax.axis_index("c")
        s = lax.axis_index("s")
        wid = c * NS + s
        row0 = s * ROWS_PER_TILE

        def _zr(r, _):
            for v in range(WP // 16):
                zrows_v[r, pl.ds(v * 16, 16)] = jnp.zeros((16,), jnp.float32)
            return 0
        lax.fori_loop(0, ZPR, _zr, 0)

        def per_tb(tb, _):
            for z in range(NZP):
                pltpu.sync_copy(
                    zrows_v, acc_sh.at[pl.ds(row0 + z * ZPR, ZPR)])
            pltpu.async_copy(sd_hbm.at[tb, 0], s_v, sm)
            pltpu.async_copy(sd_hbm.at[tb, 1], d_v, sm)
            pltpu.make_async_copy(sd_hbm.at[tb, 0], s_v, sm).wait()
            pltpu.make_async_copy(sd_hbm.at[tb, 1], d_v, sm).wait()
            plsc.subcore_barrier()

            base = tb * N
            bufs = ((srci0_v, dsti0_v, rows0_v, sg0, ss0),
                    (srci1_v, dsti1_v, rows1_v, sg1, ss1))

            def scale(ci, rows_v):
                def sc_w(w, _):
                    exv = exs_v[pl.ds(ci * CH + w * 16, 16)]
                    for k2 in range(16):
                        exb = jnp.full((16,), exv[k2], jnp.float32)
                        row = w * 16 + k2
                        for v in range(NV_SCALE):
                            rows_v[row, pl.ds(v * 16, 16)] = (
                                rows_v[row, pl.ds(v * 16, 16)] * exb)
                    return 0
                lax.fori_loop(0, CH // 16, sc_w, 0)

            def step(ci, b):
                srci_b, dsti_b, rows_b, sg_b, ss_b = bufs[b]
                srci_n, dsti_n, rows_n, sg_n, ss_n = bufs[1 - b]

                def wait_sct():
                    pltpu.make_async_copy(
                        rows_b, acc_sh.at[dsti_b], ss_b).wait()
                if isinstance(ci, int):
                    if ci >= 2:
                        wait_sct()
                else:
                    pl.when(ci >= 2)(wait_sct)

                for w in range(CH // 16):
                    esl = pl.ds(ci * CH + w * 16, 16)
                    srci_b[pl.ds(w * 16, 16)] = srcs_v[esl] + base
                    dsti_b[pl.ds(w * 16, 16)] = dsts_v[esl]
                pltpu.async_copy(hpad_hbm.at[srci_b], rows_b, sg_b)

                def fin_prev():
                    pltpu.make_async_copy(
                        hpad_hbm.at[srci_n], rows_n, sg_n).wait()
                    scale(ci - 1, rows_n)
                    pltpu.async_copy(
                        rows_n, acc_sh.at[dsti_n], ss_n, add=True)
                if isinstance(ci, int):
                    if ci >= 1:
                        fin_prev()
                else:
                    pl.when(ci >= 1)(fin_prev)

            def strip(g, _):
                e0 = wid * EW + g * SEG
                pltpu.async_copy(esrc_hbm.at[pl.ds(e0, SEG)], srcs_v, sm)
                pltpu.async_copy(edst_hbm.at[pl.ds(e0, SEG)], dsts_v, sm)
                pltpu.make_async_copy(
                    esrc_hbm.at[pl.ds(e0, SEG)], srcs_v, sm).wait()
                pltpu.make_async_copy(
                    edst_hbm.at[pl.ds(e0, SEG)], dsts_v, sm).wait()

                def p1(j, _):
                    sl = pl.ds(j * 16, 16)
                    sv = plsc.load_gather(s_v, [srcs_v[sl]])
                    dv = plsc.load_gather(d_v, [dsts_v[sl]])
                    logit = sv + dv
                    e = jnp.maximum(logit, 0.2 * logit)
                    exs_v[sl] = jnp.exp(e)
                    return 0
                lax.fori_loop(0, SEG // 16, p1, 0)

                def pair(j, _):
                    step(2 * j, 0)
                    step(2 * j + 1, 1)
                    return 0
                lax.fori_loop(0, (CPS - 1) // 2, pair, 0)
                step(CPS - 1, 0)
                pltpu.make_async_copy(
                    hpad_hbm.at[srci0_v], rows0_v, sg0).wait()
                scale(CPS - 1, rows0_v)
                pltpu.async_copy(rows0_v, acc_sh.at[dsti0_v], ss0, add=True)
                pltpu.make_async_copy(
                    rows1_v, acc_sh.at[dsti1_v], ss1).wait()
                pltpu.make_async_copy(
                    rows0_v, acc_sh.at[dsti0_v], ss0).wait()
                return 0
            lax.fori_loop(0, NSEG, strip, 0)
            plsc.subcore_barrier()

            for z in range(NCP):
                rsl = pl.ds(row0 + z * CPR, CPR)
                pltpu.sync_copy(acc_sh.at[rsl], acc_out.at[tb, c, rsl])
            return 0

        if TB == 1:
            per_tb(0, 0)
        else:
            lax.fori_loop(0, TB, per_tb, 0)

    return gat_edges


def _pad_h(h, dout):
    n = h.shape[0]
    return jnp.concatenate(
        [h, jnp.ones((n, 1), jnp.float32),
         jnp.zeros((n, WP - dout - 1), jnp.float32)], axis=1)


def _sd_of(h, a2):
    return lax.dot_general(a2, h, (((0,), (1,)), ((), ())),
                           preferred_element_type=jnp.float32,
                           precision=_PREC)


def _prep_body(act_ref, w_ref, a2_ref, hpad_ref, sd_ref):
    act = act_ref[0]
    h = jnp.dot(act, w_ref[...], preferred_element_type=jnp.float32,
                precision=_PREC)
    sd_ref[0] = _sd_of(h, a2_ref[...])
    hpad_ref[0] = _pad_h(h, H)


def _prep(act, w, a2, *, TB, din):
    return pl.pallas_call(
        _prep_body,
        grid=(TB,),
        in_specs=[
            pl.BlockSpec((1, N, din), lambda t: (t, 0, 0)),
            pl.BlockSpec((din, H), lambda t: (0, 0)),
            pl.BlockSpec((H, 2), lambda t: (0, 0)),
        ],
        out_specs=[
            pl.BlockSpec((1, N, WP), lambda t: (t, 0, 0)),
            pl.BlockSpec((1, 2, N), lambda t: (t, 0, 0)),
        ],
        out_shape=[
            jax.ShapeDtypeStruct((TB, N, WP), jnp.float32),
            jax.ShapeDtypeStruct((TB, 2, N), jnp.float32),
        ],
    )(act, w, a2)


def _act_of(acc_ref, b_ref):
    num = acc_ref[0, 0, :N] + acc_ref[0, 1, :N]
    den = num[:, H:H + 1] + 1e-16
    return jnp.maximum(num[:, :H] / den + b_ref[...], 0.0)


def _finprep_body(acc_ref, b_ref, w_ref, a2_ref, hpad_ref, sd_ref):
    act = _act_of(acc_ref, b_ref)
    h = jnp.dot(act, w_ref[...], preferred_element_type=jnp.float32,
                precision=_PREC)
    sd_ref[0] = _sd_of(h, a2_ref[...])
    hpad_ref[0] = _pad_h(h, H)


def _finprep(acc, b, w, a2, *, TB):
    return pl.pallas_call(
        _finprep_body,
        grid=(TB,),
        in_specs=[
            pl.BlockSpec((1, NC, NP, WP), lambda t: (t, 0, 0, 0)),
            pl.BlockSpec((1, H), lambda t: (0, 0)),
            pl.BlockSpec((H, H), lambda t: (0, 0)),
            pl.BlockSpec((H, 2), lambda t: (0, 0)),
        ],
        out_specs=[
            pl.BlockSpec((1, N, WP), lambda t: (t, 0, 0)),
            pl.BlockSpec((1, 2, N), lambda t: (t, 0, 0)),
        ],
        out_shape=[
            jax.ShapeDtypeStruct((TB, N, WP), jnp.float32),
            jax.ShapeDtypeStruct((TB, 2, N), jnp.float32),
        ],
    )(acc, b, w, a2)


def _finprep3_body(acc_ref, b_ref, w_ref, a2_ref, hpada_ref, hpadb_ref,
                   sd_ref):
    act = _act_of(acc_ref, b_ref)
    h = jnp.dot(act, w_ref[...], preferred_element_type=jnp.float32,
                precision=_PREC)
    sd_ref[0] = _sd_of(h, a2_ref[...])
    hpada_ref[0] = _pad_h(h[:, :H], H)
    hpadb_ref[0] = _pad_h(h[:, H:], H)


def _finprep3(acc, b, w, a2):
    return pl.pallas_call(
        _finprep3_body,
        grid=(1,),
        in_specs=[
            pl.BlockSpec((1, NC, NP, WP), lambda t: (t, 0, 0, 0)),
            pl.BlockSpec((1, H), lambda t: (0, 0)),
            pl.BlockSpec((H, F), lambda t: (0, 0)),
            pl.BlockSpec((F, 2), lambda t: (0, 0)),
        ],
        out_specs=[
            pl.BlockSpec((1, N, WP), lambda t: (t, 0, 0)),
            pl.BlockSpec((1, N, WP), lambda t: (t, 0, 0)),
            pl.BlockSpec((1, 2, N), lambda t: (t, 0, 0)),
        ],
        out_shape=[
            jax.ShapeDtypeStruct((1, N, WP), jnp.float32),
            jax.ShapeDtypeStruct((1, N, WP), jnp.float32),
            jax.ShapeDtypeStruct((1, 2, N), jnp.float32),
        ],
    )(acc, b, w, a2)


def _finpool_body(acc_ref, b_ref, pooled_ref):
    act = _act_of(acc_ref, b_ref)
    pooled_ref[0] = jnp.mean(act, axis=0, keepdims=True)


def _finpool(acc, b):
    return pl.pallas_call(
        _finpool_body,
        grid=(T,),
        in_specs=[
            pl.BlockSpec((1, NC, NP, WP), lambda t: (t, 0, 0, 0)),
            pl.BlockSpec((1, H), lambda t: (0, 0)),
        ],
        out_specs=pl.BlockSpec((1, 1, H), lambda t: (t, 0, 0)),
        out_shape=jax.ShapeDtypeStruct((T, 1, H), jnp.float32),
    )(acc, b)


def _lstm_body(pooled_ref, efw_ref, efb_ref, wih_ref, whh_ref, bih_ref,
               bhh_ref, hw_ref, hb_ref, agg_ref):
    zs = jnp.dot(pooled_ref[...], efw_ref[...],
                 preferred_element_type=jnp.float32,
                 precision=_PREC) + efb_ref[...]
    wih = wih_ref[...]
    whh = whh_ref[...]
    bsum = bih_ref[...] + bhh_ref[...]
    h = jnp.zeros((1, LH), jnp.float32)
    cst = jnp.zeros((1, LH), jnp.float32)
    for t in range(T):
        zt = zs[t:t + 1]
        gates = (lax.dot_general(zt, wih, (((1,), (1,)), ((), ())),
                                 precision=_PREC)
                 + lax.dot_general(h, whh, (((1,), (1,)), ((), ())),
                                   precision=_PREC) + bsum)
        i = jax.nn.sigmoid(gates[:, 0 * LH:1 * LH])
        f = jax.nn.sigmoid(gates[:, 1 * LH:2 * LH])
        g = jnp.tanh(gates[:, 2 * LH:3 * LH])
        o = jax.nn.sigmoid(gates[:, 3 * LH:4 * LH])
        cst = f * cst + i * g
        h = o * jnp.tanh(cst)
    agg_ref[...] = jnp.dot(h, hw_ref[...], preferred_element_type=jnp.float32,
                           precision=_PREC) + hb_ref[...]


def _lstm_head(pooled, efw, efb, wih, whh, bih, bhh, hw, hb):
    return pl.pallas_call(
        _lstm_body,
        out_shape=jax.ShapeDtypeStruct((1, L), jnp.float32),
    )(pooled, efw, efb, wih, whh, bih, bhh, hw, hb)


def _decfc_body(agg_ref, w_ref, b_ref, out_ref):
    out_ref[...] = jnp.maximum(
        jnp.dot(agg_ref[...], w_ref[...], preferred_element_type=jnp.float32,
                precision=_PREC) + b_ref[...], 0.0)


def _decfc(agg, w, b):
    BC = 32000
    G = (N * L) // BC
    return pl.pallas_call(
        _decfc_body,
        grid=(G,),
        in_specs=[
            pl.BlockSpec((1, L), lambda j: (0, 0)),
            pl.BlockSpec((L, BC), lambda j: (0, j)),
            pl.BlockSpec((1, BC), lambda j: (0, j)),
        ],
        out_specs=pl.BlockSpec((1, BC), lambda j: (0, j)),
        out_shape=jax.ShapeDtypeStruct((1, N * L), jnp.float32),
    )(agg, w, b)


def _final_body(acca_ref, accb_ref, b_ref, out_ref):
    numa = acca_ref[0, 0, :N] + acca_ref[0, 1, :N]
    dena = numa[:, H:H + 1] + 1e-16
    numb = accb_ref[0, 0, :N] + accb_ref[0, 1, :N]
    denb = numb[:, H:H + 1] + 1e-16
    out_ref[...] = jnp.concatenate(
        [numa[:, :H] / dena, numb[:, :H] / denb], axis=1) + b_ref[...]


def _final(acca, accb, b):
    return pl.pallas_call(
        _final_body,
        out_shape=jax.ShapeDtypeStruct((N, F), jnp.float32),
    )(acca, accb, b)


def _a2(p):
    return jnp.stack([p["a_src"], p["a_dst"]], axis=1)


def kernel(x, edge_index, params):
    esrc = edge_index[0]
    edst = edge_index[1]
    enc0, enc1 = params["enc"]
    dec = params["dec"]

    hpad, sd = _prep(x, enc0["W"], _a2(enc0), TB=T, din=F)
    acc = _make_gat_edges(T)(hpad.reshape(T * N, WP), sd, esrc, edst)

    hpad, sd = _finprep(acc, enc0["b"].reshape(1, H), enc1["W"], _a2(enc1),
                        TB=T)
    acc = _make_gat_edges(T)(hpad.reshape(T * N, WP), sd, esrc, edst)

    pooled = _finpool(acc, enc1["b"].reshape(1, H)).reshape(T, H)

    agg = _lstm_head(
        pooled, params["enc_fc_W"], params["enc_fc_b"].reshape(1, L),
        params["lstm_W_ih"], params["lstm_W_hh"],
        params["lstm_b_ih"].reshape(1, 4 * LH),
        params["lstm_b_hh"].reshape(1, 4 * LH),
        params["head_W"], params["head_b"].reshape(1, L))

    dec_in = _decfc(agg, params["dec_fc_W"],
                    params["dec_fc_b"].reshape(1, N * L)).reshape(1, N, L)

    hpad, sd = _prep(dec_in, dec[0]["W"], _a2(dec[0]), TB=1, din=L)
    acc = _make_gat_edges(1)(hpad.reshape(N, WP), sd, esrc, edst)

    hpad, sd = _finprep(acc, dec[0]["b"].reshape(1, H), dec[1]["W"],
                        _a2(dec[1]), TB=1)
    acc = _make_gat_edges(1)(hpad.reshape(N, WP), sd, esrc, edst)

    hpad, sd = _finprep(acc, dec[1]["b"].reshape(1, H), dec[2]["W"],
                        _a2(dec[2]), TB=1)
    acc = _make_gat_edges(1)(hpad.reshape(N, WP), sd, esrc, edst)

    hpada, hpadb, sd = _finprep3(acc, dec[2]["b"].reshape(1, H), dec[3]["W"],
                                 _a2(dec[3]))
    acca = _make_gat_edges(1)(hpada.reshape(N, WP), sd, esrc, edst)
    accb = _make_gat_edges(1)(hpadb.reshape(N, WP), sd, esrc, edst)

    return _final(acca, accb, dec[3]["b"].reshape(1, F))
